# baseline (speedup 1.0000x reference)
import numpy as np
import jax
import jax.numpy as jnp
from jax import lax
from jax.experimental import pallas as pl
from jax.experimental.pallas import tpu as pltpu
from jax.experimental.pallas import tpu_sc as plsc

S = 197
N = S * S
HALF = 384
JC = 48
NBUF = 3

_info = plsc.get_sparse_core_info()
_NC, _NS = _info.num_cores, _info.num_subcores
NW = _NC * _NS
NCH = -(-N // (JC * NW))
TCH = NCH * NW
_starts = np.minimum(np.arange(TCH) * JC, N - JC)


def _body(ct_hbm, ij_hbm, out_hbm, ij_v, b0, b1, b2, g0, g1, g2, w0, w1, w2):
    wid = lax.axis_index("s") * _NC + lax.axis_index("c")

    pltpu.sync_copy(ij_hbm.at[pl.ds(wid * NCH, NCH)], ij_v)

    buf = (b0, b1, b2)
    gsem, wsem = (g0, g1, g2), (w0, w1, w2)

    def start(t):
        c = wid * NCH + t
        return jnp.minimum(c * JC, N - JC)

    def gather(t):
        p = t % NBUF
        return pltpu.make_async_copy(ct_hbm.at[ij_v.at[t]], buf[p], gsem[p])

    def write(t):
        p = t % NBUF
        return pltpu.make_async_copy(
            buf[p], out_hbm.at[pl.ds(2 * start(t), 2 * JC)], wsem[p])

    gather(0).start()
    for t in range(NCH):
        if t >= NBUF - 1:
            write(t - (NBUF - 1)).wait()
        if t + 1 < NCH:
            gather(t + 1).start()
        gather(t).wait()
        write(t).start()
    for q in range(NBUF - 1, 0, -1):
        write(NCH - q).wait()


def kernel(x_table, y_table, x_dis, y_dis):
    rows = x_table.shape[0]
    ct = jnp.concatenate([x_table, y_table], axis=0)
    ct_rep = jnp.tile(ct, (NW, 1))

    f = jnp.stack([x_dis.reshape(N), y_dis.reshape(N) + rows],
                  axis=-1).reshape(2 * N)
    pos = 2 * _starts[:, None] + np.arange(2 * JC)[None, :]
    owner_off = ((np.arange(TCH) // NCH) * 2 * rows).astype(np.int32)
    ij = f[pos] + owner_off[:, None]

    run = pl.kernel(
        _body,
        out_type=jax.ShapeDtypeStruct((2 * N, HALF), jnp.float32),
        compiler_params=pltpu.CompilerParams(use_tc_tiling_on_sc=False),
        mesh=plsc.VectorSubcoreMesh(core_axis_name="c", subcore_axis_name="s"),
        scratch_types=[
            pltpu.VMEM((NCH, 2 * JC), jnp.int32),
            pltpu.VMEM((2 * JC, HALF), jnp.float32),
            pltpu.VMEM((2 * JC, HALF), jnp.float32),
            pltpu.VMEM((2 * JC, HALF), jnp.float32),
            pltpu.SemaphoreType.DMA,
            pltpu.SemaphoreType.DMA,
            pltpu.SemaphoreType.DMA,
            pltpu.SemaphoreType.DMA,
            pltpu.SemaphoreType.DMA,
            pltpu.SemaphoreType.DMA,
        ],
    )
    return run(ct_rep, ij).reshape(S, S, 2 * HALF)


# speedup vs baseline: 1.0134x; 1.0134x over previous
import numpy as np
import jax
import jax.numpy as jnp
from jax import lax
from jax.experimental import pallas as pl
from jax.experimental.pallas import tpu as pltpu
from jax.experimental.pallas import tpu_sc as plsc

S = 197
N = S * S
HALF = 384
JC = 64
NBUF = 2

_info = plsc.get_sparse_core_info()
_NC, _NS = _info.num_cores, _info.num_subcores
NW = _NC * _NS
NCH = -(-N // (JC * NW))
TCH = NCH * NW
_starts = np.minimum(np.arange(TCH) * JC, N - JC)


def _body(ct_hbm, ij_hbm, out_hbm, ij_v, b0, b1, g0, g1, w0, w1):
    wid = lax.axis_index("s") * _NC + lax.axis_index("c")

    pltpu.sync_copy(ij_hbm.at[pl.ds(wid * NCH, NCH)], ij_v)

    buf = (b0, b1)
    gsem, wsem = (g0, g1), (w0, w1)

    def start(t):
        c = wid * NCH + t
        return jnp.minimum(c * JC, N - JC)

    def gather(t):
        p = t % NBUF
        return pltpu.make_async_copy(ct_hbm.at[ij_v.at[t]], buf[p], gsem[p])

    def write(t):
        p = t % NBUF
        return pltpu.make_async_copy(
            buf[p], out_hbm.at[pl.ds(2 * start(t), 2 * JC)], wsem[p])

    gather(0).start()
    for t in range(NCH):
        if t >= NBUF - 1:
            write(t - (NBUF - 1)).wait()
        if t + 1 < NCH:
            gather(t + 1).start()
        gather(t).wait()
        write(t).start()
    for q in range(NBUF - 1, 0, -1):
        write(NCH - q).wait()


def kernel(x_table, y_table, x_dis, y_dis):
    rows = x_table.shape[0]
    ct = jnp.concatenate([x_table, y_table], axis=0)
    ct_rep = jnp.tile(ct, (NW, 1))

    f = jnp.stack([x_dis.reshape(N), y_dis.reshape(N) + rows],
                  axis=-1).reshape(2 * N)
    pos = 2 * _starts[:, None] + np.arange(2 * JC)[None, :]
    owner_off = ((np.arange(TCH) // NCH) * 2 * rows).astype(np.int32)
    ij = f[pos] + owner_off[:, None]

    run = pl.kernel(
        _body,
        out_type=jax.ShapeDtypeStruct((2 * N, HALF), jnp.float32),
        compiler_params=pltpu.CompilerParams(use_tc_tiling_on_sc=False),
        mesh=plsc.VectorSubcoreMesh(core_axis_name="c", subcore_axis_name="s"),
        scratch_types=[
            pltpu.VMEM((NCH, 2 * JC), jnp.int32),
            pltpu.VMEM((2 * JC, HALF), jnp.float32),
            pltpu.VMEM((2 * JC, HALF), jnp.float32),
            pltpu.SemaphoreType.DMA,
            pltpu.SemaphoreType.DMA,
            pltpu.SemaphoreType.DMA,
            pltpu.SemaphoreType.DMA,
        ],
    )
    return run(ct_rep, ij).reshape(S, S, 2 * HALF)
